# speedup vs baseline: 2.5380x; 2.5380x over previous
import functools
import math

import jax
import jax.numpy as jnp
from jax import lax
from jax.experimental import pallas as pl
from jax.experimental.pallas import tpu as pltpu
from jax.experimental.pallas import tpu_sc as plsc

_VOCAB = 1000
_D = 128
_SCALE = math.sqrt(float(_D))

_NC = 2
_NS = 16
_NW = _NC * _NS

_CHUNK = 128


def _scale_table_body(w_ref, o_ref):
    o_ref[...] = w_ref[...] * _SCALE


def _scale_table(W):
    return pl.pallas_call(
        _scale_table_body,
        out_shape=jax.ShapeDtypeStruct(W.shape, W.dtype),
    )(W)


def _make_sc_gather(n_idx):
    assert n_idx % (_NW * _CHUNK) == 0
    rows_per_w = n_idx // (_NW * _CHUNK)
    b_per_w = n_idx // _NW

    mesh = plsc.VectorSubcoreMesh(core_axis_name="c", subcore_axis_name="s")

    @functools.partial(
        pl.kernel,
        mesh=mesh,
        out_type=jax.ShapeDtypeStruct((n_idx, _D), jnp.float32),
        scratch_types=[
            pltpu.VMEM((rows_per_w, _CHUNK), jnp.int32),
            pltpu.VMEM((_CHUNK, _D), jnp.float32),
            pltpu.SemaphoreType.DMA,
        ],
    )
    def k(table_hbm, idx_hbm, out_hbm, idx_v, rows_v, sem):
        wid = lax.axis_index("s") * _NC + lax.axis_index("c")
        pltpu.sync_copy(idx_hbm.at[pl.ds(wid * rows_per_w, rows_per_w)], idx_v)
        out_base = wid * b_per_w

        def body(c, carry):
            pltpu.async_copy(table_hbm.at[idx_v.at[c]], rows_v, sem).wait()
            pltpu.sync_copy(rows_v, out_hbm.at[pl.ds(out_base + c * _CHUNK, _CHUNK)])
            return carry

        lax.fori_loop(0, rows_per_w, body, 0)

    return k


def kernel(x, W):
    batch, hist = x.shape
    n_idx = batch * hist
    Ws = _scale_table(W)
    idx2d = x.reshape(n_idx // _CHUNK, _CHUNK)
    out = _make_sc_gather(n_idx)(Ws, idx2d)
    return out.reshape(batch, hist, _D)


# speedup vs baseline: 2.6461x; 1.0426x over previous
import functools
import math

import jax
import jax.numpy as jnp
from jax import lax
from jax.experimental import pallas as pl
from jax.experimental.pallas import tpu as pltpu
from jax.experimental.pallas import tpu_sc as plsc

_VOCAB = 1000
_D = 128
_SCALE = math.sqrt(float(_D))

_NC = 2
_NS = 16
_NW = _NC * _NS

_CHUNK = 128
_NBUF = 4


def _scale_table_body(w_ref, o_ref):
    o_ref[...] = w_ref[...] * _SCALE


def _scale_table(W):
    return pl.pallas_call(
        _scale_table_body,
        out_shape=jax.ShapeDtypeStruct(W.shape, W.dtype),
    )(W)


def _make_sc_gather(n_idx):
    assert n_idx % (_NW * _CHUNK) == 0
    rows_per_w = n_idx // (_NW * _CHUNK)
    assert rows_per_w % _NBUF == 0
    b_per_w = n_idx // _NW

    mesh = plsc.VectorSubcoreMesh(core_axis_name="c", subcore_axis_name="s")

    @functools.partial(
        pl.kernel,
        mesh=mesh,
        out_type=jax.ShapeDtypeStruct((n_idx, _D), jnp.float32),
        scratch_types=[
            pltpu.VMEM((rows_per_w, _CHUNK), jnp.int32),
            pltpu.VMEM((_NBUF, _CHUNK, _D), jnp.float32),
        ]
        + [pltpu.SemaphoreType.DMA] * (2 * _NBUF),
    )
    def k(table_hbm, idx_hbm, out_hbm, idx_v, rows_v, *sems):
        gsems, ssems = sems[:_NBUF], sems[_NBUF:]
        wid = lax.axis_index("s") * _NC + lax.axis_index("c")
        pltpu.sync_copy(idx_hbm.at[pl.ds(wid * rows_per_w, rows_per_w)], idx_v)
        out_base = wid * b_per_w

        def gather(c, b):
            return pltpu.make_async_copy(
                table_hbm.at[idx_v.at[c]], rows_v.at[b], gsems[b])

        def scatter(c, b):
            return pltpu.make_async_copy(
                rows_v.at[b],
                out_hbm.at[pl.ds(out_base + c * _CHUNK, _CHUNK)],
                ssems[b])

        for b in range(_NBUF):
            gather(b, b).start()

        def body(g, carry):
            for b in range(_NBUF):
                c = g * _NBUF + b
                gather(c, b).wait()
                sc = scatter(c, b)
                sc.start()
                sc.wait()
                gather(c + _NBUF, b).start()
            return carry

        lax.fori_loop(0, rows_per_w // _NBUF - 1, body, 0)

        for b in range(_NBUF):
            c = rows_per_w - _NBUF + b
            gather(c, b).wait()
            sc = scatter(c, b)
            sc.start()
            sc.wait()

    return k


def kernel(x, W):
    batch, hist = x.shape
    n_idx = batch * hist
    Ws = _scale_table(W)
    idx2d = x.reshape(n_idx // _CHUNK, _CHUNK)
    out = _make_sc_gather(n_idx)(Ws, idx2d)
    return out.reshape(batch, hist, _D)


# speedup vs baseline: 4.3377x; 1.6393x over previous
import functools
import math

import jax
import jax.numpy as jnp
from jax import lax
from jax.experimental import pallas as pl
from jax.experimental.pallas import tpu as pltpu
from jax.experimental.pallas import tpu_sc as plsc

_VOCAB = 1000
_D = 128
_SCALE = math.sqrt(float(_D))

_NC = 2
_NS = 16
_NW = _NC * _NS

_CHUNK = 128
_NBUF = 4


def _scale_table_body(w_ref, o_ref):
    o_ref[...] = w_ref[...] * _SCALE


def _scale_table(W):
    return pl.pallas_call(
        _scale_table_body,
        out_shape=jax.ShapeDtypeStruct(W.shape, W.dtype),
    )(W)


def _make_sc_gather(batch, hist):
    assert batch % _NW == 0
    b_per_w = batch // _NW
    assert b_per_w % _CHUNK == 0
    ncb = b_per_w // _CHUNK
    nchunks = hist * ncb
    assert nchunks % _NBUF == 0

    mesh = plsc.VectorSubcoreMesh(core_axis_name="c", subcore_axis_name="s")

    @functools.partial(
        pl.kernel,
        mesh=mesh,
        out_type=jax.ShapeDtypeStruct((batch, hist, _D), jnp.float32),
        scratch_types=[
            pltpu.VMEM((hist, b_per_w), jnp.int32),
            pltpu.VMEM((_NBUF, _CHUNK, _D), jnp.float32),
        ]
        + [pltpu.SemaphoreType.DMA] * (2 * _NBUF),
    )
    def k(table_hbm, idxT_hbm, out_hbm, idx_v, rows_v, *sems):
        gsems, ssems = sems[:_NBUF], sems[_NBUF:]
        wid = lax.axis_index("s") * _NC + lax.axis_index("c")
        wb = wid * b_per_w
        pltpu.sync_copy(idxT_hbm.at[:, pl.ds(wb, b_per_w)], idx_v)

        def gather(c, b):
            j, cb = c // ncb, c % ncb
            return pltpu.make_async_copy(
                table_hbm.at[idx_v.at[j, pl.ds(cb * _CHUNK, _CHUNK)]],
                rows_v.at[b], gsems[b])

        def scatter(c, b):
            j, cb = c // ncb, c % ncb
            return pltpu.make_async_copy(
                rows_v.at[b],
                out_hbm.at[pl.ds(wb + cb * _CHUNK, _CHUNK), j],
                ssems[b])

        for b in range(_NBUF):
            gather(b, b).start()

        def body(g, carry):
            for b in range(_NBUF):
                c = g * _NBUF + b
                gather(c, b).wait()
                sc = scatter(c, b)
                sc.start()
                sc.wait()
                gather(c + _NBUF, b).start()
            return carry

        lax.fori_loop(0, nchunks // _NBUF - 1, body, 0)

        for b in range(_NBUF):
            c = nchunks - _NBUF + b
            gather(c, b).wait()
            sc = scatter(c, b)
            sc.start()
            sc.wait()

    return k


def kernel(x, W):
    batch, hist = x.shape
    Ws = _scale_table(W)
    return _make_sc_gather(batch, hist)(Ws, x.T)


# speedup vs baseline: 4.3413x; 1.0008x over previous
import functools
import math

import jax
import jax.numpy as jnp
from jax import lax
from jax.experimental import pallas as pl
from jax.experimental.pallas import tpu as pltpu
from jax.experimental.pallas import tpu_sc as plsc

_VOCAB = 1000
_D = 128
_SCALE = math.sqrt(float(_D))

_NC = 2
_NS = 16
_NW = _NC * _NS

_CHUNK = 128
_NBUF = 4


def _scale_table_body(w_ref, o_ref):
    o_ref[...] = w_ref[...] * _SCALE


def _scale_table(W):
    return pl.pallas_call(
        _scale_table_body,
        out_shape=jax.ShapeDtypeStruct(W.shape, W.dtype),
    )(W)


def _make_sc_gather(batch, hist):
    assert batch % _NW == 0
    b_per_w = batch // _NW
    assert b_per_w % _CHUNK == 0
    ncb = b_per_w // _CHUNK
    nchunks = hist * ncb
    assert nchunks % _NBUF == 0

    mesh = plsc.VectorSubcoreMesh(core_axis_name="c", subcore_axis_name="s")

    @functools.partial(
        pl.kernel,
        mesh=mesh,
        out_type=jax.ShapeDtypeStruct((batch, hist, _D), jnp.float32),
        scratch_types=[
            pltpu.VMEM((hist, b_per_w), jnp.int32),
            pltpu.VMEM((_NBUF, _CHUNK, _D), jnp.float32),
        ]
        + [pltpu.SemaphoreType.DMA] * (2 * _NBUF),
        compiler_params=pltpu.CompilerParams(use_tc_tiling_on_sc=True),
    )
    def k(table_hbm, idxT_hbm, out_hbm, idx_v, rows_v, *sems):
        gsems, ssems = sems[:_NBUF], sems[_NBUF:]
        wid = lax.axis_index("s") * _NC + lax.axis_index("c")
        wb = wid * b_per_w
        pltpu.sync_copy(idxT_hbm.at[:, pl.ds(wb, b_per_w)], idx_v)

        def gather(c, b):
            j, cb = c // ncb, c % ncb
            return pltpu.make_async_copy(
                table_hbm.at[idx_v.at[j, pl.ds(cb * _CHUNK, _CHUNK)]],
                rows_v.at[b], gsems[b])

        def scatter(c, b):
            j, cb = c // ncb, c % ncb
            return pltpu.make_async_copy(
                rows_v.at[b],
                out_hbm.at[pl.ds(wb + cb * _CHUNK, _CHUNK), j],
                ssems[b])

        for b in range(_NBUF):
            gather(b, b).start()

        def body(g, carry):
            for b in range(_NBUF):
                c = g * _NBUF + b
                gather(c, b).wait()
                sc = scatter(c, b)
                sc.start()
                sc.wait()
                gather(c + _NBUF, b).start()
            return carry

        lax.fori_loop(0, nchunks // _NBUF - 1, body, 0)

        for b in range(_NBUF):
            c = nchunks - _NBUF + b
            gather(c, b).wait()
            sc = scatter(c, b)
            sc.start()
            sc.wait()

    return k


def kernel(x, W):
    batch, hist = x.shape
    Ws = _scale_table(W)
    return _make_sc_gather(batch, hist)(Ws, x.T)


# speedup vs baseline: 6.6931x; 1.5417x over previous
import functools
import math

import jax
import jax.numpy as jnp
from jax import lax
from jax.experimental import pallas as pl
from jax.experimental.pallas import tpu as pltpu
from jax.experimental.pallas import tpu_sc as plsc

_VOCAB = 1000
_D = 128
_SCALE = math.sqrt(float(_D))

_NC = 2
_NS = 16
_NW = _NC * _NS

_CHUNK = 128
_NBUF = 4


def _scale_table_body(w_ref, o_ref):
    o_ref[...] = w_ref[...] * _SCALE


def _scale_table(W):
    return pl.pallas_call(
        _scale_table_body,
        out_shape=jax.ShapeDtypeStruct(W.shape, W.dtype),
    )(W)


def _make_sc_gather(batch, hist):
    assert batch % _NW == 0
    b_per_w = batch // _NW
    assert b_per_w % _CHUNK == 0
    ncb = b_per_w // _CHUNK
    nchunks = hist * ncb
    assert nchunks % _NBUF == 0

    mesh = plsc.VectorSubcoreMesh(core_axis_name="c", subcore_axis_name="s")

    @functools.partial(
        pl.kernel,
        mesh=mesh,
        out_type=jax.ShapeDtypeStruct((hist, batch, _D), jnp.float32),
        scratch_types=[
            pltpu.VMEM((hist, b_per_w), jnp.int32),
            pltpu.VMEM((_NBUF, _CHUNK, _D), jnp.float32),
        ]
        + [pltpu.SemaphoreType.DMA] * (2 * _NBUF),
    )
    def k(table_hbm, idxT_hbm, out_hbm, idx_v, rows_v, *sems):
        gsems, ssems = sems[:_NBUF], sems[_NBUF:]
        wid = lax.axis_index("s") * _NC + lax.axis_index("c")
        wb = wid * b_per_w
        pltpu.sync_copy(idxT_hbm.at[:, pl.ds(wb, b_per_w)], idx_v)

        def gather(c, b):
            j, cb = c // ncb, c % ncb
            return pltpu.make_async_copy(
                table_hbm.at[idx_v.at[j, pl.ds(cb * _CHUNK, _CHUNK)]],
                rows_v.at[b], gsems[b])

        def scatter(c, b):
            j, cb = c // ncb, c % ncb
            return pltpu.make_async_copy(
                rows_v.at[b],
                out_hbm.at[j, pl.ds(wb + cb * _CHUNK, _CHUNK)],
                ssems[b])

        for b in range(_NBUF):
            gather(b, b).start()

        def body(g, carry):
            for b in range(_NBUF):
                c = g * _NBUF + b
                gather(c, b).wait()
                sc = scatter(c, b)
                sc.start()
                sc.wait()
                gather(c + _NBUF, b).start()
            return carry

        lax.fori_loop(0, nchunks // _NBUF - 1, body, 0)

        for b in range(_NBUF):
            c = nchunks - _NBUF + b
            gather(c, b).wait()
            sc = scatter(c, b)
            sc.start()
            sc.wait()

    return k


def kernel(x, W):
    batch, hist = x.shape
    Ws = _scale_table(W)
    outT = _make_sc_gather(batch, hist)(Ws, x.T)
    return jnp.transpose(outT, (1, 0, 2))


# speedup vs baseline: 16.0413x; 2.3967x over previous
import functools
import math

import jax
import jax.numpy as jnp
from jax import lax
from jax.experimental import pallas as pl
from jax.experimental.pallas import tpu as pltpu
from jax.experimental.pallas import tpu_sc as plsc

_VOCAB = 1000
_D = 128
_SCALE = math.sqrt(float(_D))

_NC = 2
_NS = 16
_NW = _NC * _NS

_CHUNK = 128
_NBUF = 4


_VPAD = 1024


def _scale_table_body(w_ref, o_ref):
    o_ref[pl.ds(0, _VOCAB), :] = w_ref[...] * _SCALE


def _scale_table(W):
    return pl.pallas_call(
        _scale_table_body,
        out_shape=jax.ShapeDtypeStruct((_VPAD, _D), W.dtype),
    )(W)


def _make_sc_gather(batch, hist):
    assert batch % _NW == 0
    b_per_w = batch // _NW
    assert b_per_w % _CHUNK == 0
    ncb = b_per_w // _CHUNK
    nchunks = hist * ncb
    assert nchunks % _NBUF == 0

    mesh = plsc.VectorSubcoreMesh(core_axis_name="c", subcore_axis_name="s")

    @functools.partial(
        pl.kernel,
        mesh=mesh,
        out_type=jax.ShapeDtypeStruct((hist, batch, _D), jnp.float32),
        scratch_types=[
            pltpu.VMEM((hist, b_per_w), jnp.int32),
            pltpu.VMEM((_NBUF, _CHUNK, _D), jnp.float32),
            pltpu.VMEM_SHARED((_VPAD, _D), jnp.float32),
        ]
        + [pltpu.SemaphoreType.DMA] * (2 * _NBUF),
    )
    def k(table_hbm, idxT_hbm, out_hbm, idx_v, rows_v, table_sh, *sems):
        gsems, ssems = sems[:_NBUF], sems[_NBUF:]
        sid = lax.axis_index("s")
        wid = sid * _NC + lax.axis_index("c")
        wb = wid * b_per_w
        stage = _VPAD // _NS
        pltpu.sync_copy(table_hbm.at[pl.ds(sid * stage, stage)],
                        table_sh.at[pl.ds(sid * stage, stage)])
        pltpu.sync_copy(idxT_hbm.at[:, pl.ds(wb, b_per_w)], idx_v)
        plsc.subcore_barrier()

        def gather(c, b):
            j, cb = c // ncb, c % ncb
            return pltpu.make_async_copy(
                table_sh.at[idx_v.at[j, pl.ds(cb * _CHUNK, _CHUNK)]],
                rows_v.at[b], gsems[b])

        def scatter(c, b):
            j, cb = c // ncb, c % ncb
            return pltpu.make_async_copy(
                rows_v.at[b],
                out_hbm.at[j, pl.ds(wb + cb * _CHUNK, _CHUNK)],
                ssems[b])

        for b in range(_NBUF):
            gather(b, b).start()

        def body(g, carry):
            for b in range(_NBUF):
                c = g * _NBUF + b
                gather(c, b).wait()
                scatter(c, b).start()
            for b in range(_NBUF):
                c = g * _NBUF + b
                scatter(c, b).wait()
                gather(c + _NBUF, b).start()
            return carry

        lax.fori_loop(0, nchunks // _NBUF - 1, body, 0)

        for b in range(_NBUF):
            c = nchunks - _NBUF + b
            gather(c, b).wait()
            scatter(c, b).start()
        for b in range(_NBUF):
            c = nchunks - _NBUF + b
            scatter(c, b).wait()

    return k


def kernel(x, W):
    batch, hist = x.shape
    Ws = _scale_table(W)
    outT = _make_sc_gather(batch, hist)(Ws, x.T)
    return jnp.transpose(outT, (1, 0, 2))
